# baseline (speedup 1.0000x reference)
import jax
import jax.numpy as jnp
from jax.experimental import pallas as pl
from jax.experimental.pallas import tpu as pltpu

_BM = 80
_NSTREAM = 5


def _gcn_block(*refs):
    a_refs = refs[:_NSTREAM]
    e_ref, o_ref = refs[_NSTREAM], refs[_NSTREAM + 1]
    e16 = e_ref[...].astype(jnp.bfloat16)
    for s in range(_NSTREAM):
        o_ref[s * _BM:(s + 1) * _BM, :] = jnp.dot(
            a_refs[s][...].astype(jnp.bfloat16), e16,
            preferred_element_type=jnp.float32)


def kernel(A_hat, E):
    n, k = A_hat.shape
    d = E.shape[1]
    rows_per_step = _BM * _NSTREAM
    in_specs = [
        pl.BlockSpec((_BM, k), lambda i, s=s: (_NSTREAM * i + s, 0))
        for s in range(_NSTREAM)
    ]
    in_specs.append(pl.BlockSpec((k, d), lambda i: (0, 0)))
    return pl.pallas_call(
        _gcn_block,
        grid=(n // rows_per_step,),
        in_specs=in_specs,
        out_specs=pl.BlockSpec((rows_per_step, d), lambda i: (i, 0)),
        out_shape=jax.ShapeDtypeStruct((n, d), jnp.float32),
        compiler_params=pltpu.CompilerParams(
            dimension_semantics=("arbitrary",),
        ),
    )(*([A_hat] * _NSTREAM), E)


# speedup vs baseline: 1.0224x; 1.0224x over previous
import jax
import jax.numpy as jnp
from jax.experimental import pallas as pl
from jax.experimental.pallas import tpu as pltpu

_BM = 80
_NBUF = 5


def _gcn_body(a_hbm, e_ref, o_ref, a_buf, sems):
    nblk = a_hbm.shape[0] // _BM
    nrot = nblk // _NBUF

    def copy(slot, idx):
        return pltpu.make_async_copy(
            a_hbm.at[pl.ds(idx * _BM, _BM), :],
            a_buf.at[slot],
            sems.at[slot],
        )

    def stage(slot, idx):
        copy(slot, idx).wait()
        o_ref[pl.ds(idx * _BM, _BM), :] = jnp.dot(
            a_buf[slot], e_ref[...], preferred_element_type=jnp.float32)

    for s in range(_NBUF - 1):
        copy(s, s).start()

    def rotation(i, carry):
        base = i * _NBUF
        for s in range(_NBUF):
            copy((s + _NBUF - 1) % _NBUF, base + s + _NBUF - 1).start()
            stage(s, base + s)
        return carry

    jax.lax.fori_loop(0, nrot - 1, rotation, 0)
    base = (nrot - 1) * _NBUF
    for s in range(_NBUF):
        if s == 0:
            copy(_NBUF - 1, base + _NBUF - 1).start()
        stage(s, base + s)


def kernel(A_hat, E):
    n, k = A_hat.shape
    d = E.shape[1]
    return pl.pallas_call(
        _gcn_body,
        in_specs=[
            pl.BlockSpec(memory_space=pltpu.MemorySpace.HBM),
            pl.BlockSpec(memory_space=pltpu.MemorySpace.VMEM),
        ],
        out_specs=pl.BlockSpec(memory_space=pltpu.MemorySpace.VMEM),
        out_shape=jax.ShapeDtypeStruct((n, d), jnp.float32),
        scratch_shapes=[
            pltpu.MemorySpace.VMEM((_NBUF, _BM, k), jnp.float32),
            pltpu.SemaphoreType.DMA((_NBUF,)),
        ],
    )(A_hat, E)
